# baseline (speedup 1.0000x reference)
import functools

import jax
import jax.numpy as jnp
from jax import lax
from jax.experimental import pallas as pl
from jax.experimental.pallas import tpu as pltpu
from jax.experimental.pallas import tpu_sc as plsc

N = 10000
D = 128
H = 16
C = 7
E = 320000

NC = 2
NS = 16
NW = NC * NS

NPAD = 10240
ROWS_PT = NPAD // NS
NBLK = ROWS_PT // 16

EPT = E // NW
G = 2000
NGR = EPT // G
W2W = 8


@functools.cache
def _mesh():
    return plsc.VectorSubcoreMesh(
        core_axis_name="c", subcore_axis_name="s", num_cores=NC, num_subcores=NS
    )


def _newton_rsqrt(x):
    i = plsc.bitcast(x, jnp.int32)
    i = 0x5F3759DF - lax.shift_right_logical(i, 1)
    y = plsc.bitcast(i, jnp.float32)
    for _ in range(3):
        y = y * (1.5 - 0.5 * x * y * y)
    return y


def _agg_pipeline(gsrc_sh, agg_sh, idx_s, idx_d, rows, gsem, ssem0, ssem1):
    def ssem_wait(parity_is_odd, j):
        @pl.when(parity_is_odd == 0)
        def _():
            pltpu.make_async_copy(
                rows.at[0], agg_sh.at[idx_d.at[j]], ssem0).wait()

        @pl.when(parity_is_odd == 1)
        def _():
            pltpu.make_async_copy(
                rows.at[1], agg_sh.at[idx_d.at[j]], ssem1).wait()

    pltpu.async_copy(gsrc_sh.at[idx_s.at[0]], rows.at[0], gsem)

    def step(j, carry):
        buf = lax.rem(j, 2)
        pltpu.make_async_copy(
            gsrc_sh.at[idx_s.at[j]], rows.at[buf], gsem).wait()

        @pl.when(buf == 0)
        def _():
            pltpu.async_copy(
                rows.at[0], agg_sh.at[idx_d.at[j]], ssem0, add=True)

        @pl.when(buf == 1)
        def _():
            pltpu.async_copy(
                rows.at[1], agg_sh.at[idx_d.at[j]], ssem1, add=True)

        @pl.when(j < NGR - 1)
        def _():
            nbuf = lax.rem(j + 1, 2)

            @pl.when(j >= 1)
            def _():
                ssem_wait(nbuf, j - 1)

            pltpu.async_copy(gsrc_sh.at[idx_s.at[j + 1]], rows.at[nbuf], gsem)

        return carry

    lax.fori_loop(0, NGR, step, 0)
    ssem_wait(lax.rem(NGR - 2, 2), NGR - 2)
    ssem_wait(lax.rem(NGR - 1, 2), NGR - 1)


def _sc1_body(h1_hbm, pk_hbm,
              agg_out, dinv_out,
              pk2, idx_s, ones_v, h1_v, dinv_v, rows,
              deg_sh, g1_sh, agg_sh,
              hsem, dsem, gsem, ssem0, ssem1):
    c = lax.axis_index("c")
    s = lax.axis_index("s")
    w = c * NS + s
    base = s * ROWS_PT

    def zero_blk(b, carry):
        dinv_v[pl.ds(b * 16, 16)] = jnp.zeros((16,), jnp.float32)
        for k in range(16):
            h1_v[b * 16 + k, :] = jnp.zeros((16,), jnp.float32)
        return carry

    lax.fori_loop(0, NBLK, zero_blk, 0)
    pltpu.sync_copy(h1_v, agg_sh.at[pl.ds(base, ROWS_PT)])
    pltpu.sync_copy(dinv_v, deg_sh.at[pl.ds(base, ROWS_PT)])

    pltpu.async_copy(h1_hbm.at[pl.ds(base, ROWS_PT)], h1_v, hsem)
    pltpu.sync_copy(pk_hbm.at[s], pk2.at[0])
    pltpu.sync_copy(pk_hbm.at[NS + s], pk2.at[1])
    for i in range(G // 16):
        ones_v[pl.ds(i * 16, 16)] = jnp.ones((16,), jnp.float32)

    def unpack_blk(i, carry):
        sl = pl.ds(i * 16, 16)
        for ch in range(2):
            for j in range(NGR):
                v = pk2[ch, j, sl]

                @pl.when(c == ch)
                def _(v=v, j=j, sl=sl):
                    idx_s[j, sl] = lax.shift_right_logical(v, 14)

                pk2[ch, j, sl] = v & 16383
        return carry

    lax.fori_loop(0, G // 16, unpack_blk, 0)

    plsc.subcore_barrier()

    descs = []
    for chunk in range(2):
        for j in range(NGR):
            descs.append(pltpu.async_copy(
                ones_v, deg_sh.at[pk2.at[chunk, j]], dsem, add=True))
    for dsc in descs:
        dsc.wait()
    plsc.subcore_barrier()

    pltpu.sync_copy(deg_sh.at[pl.ds(base, ROWS_PT)], dinv_v)

    def rsqrt_blk(b, carry):
        x = dinv_v[pl.ds(b * 16, 16)] + 1.0
        dinv_v[pl.ds(b * 16, 16)] = _newton_rsqrt(x)
        return carry

    lax.fori_loop(0, NBLK, rsqrt_blk, 0)

    pltpu.make_async_copy(h1_hbm.at[pl.ds(base, ROWS_PT)], h1_v, hsem).wait()

    def scale_blk(b, carry):
        dvec = dinv_v[pl.ds(b * 16, 16)]
        for k in range(16):
            i = b * 16 + k
            h1_v[i, :] = h1_v[i, :] * dvec[k]
        return carry

    lax.fori_loop(0, NBLK, scale_blk, 0)

    pltpu.sync_copy(h1_v, g1_sh.at[pl.ds(base, ROWS_PT)])

    @pl.when(c == 0)
    def _():
        pltpu.sync_copy(dinv_v, dinv_out.at[pl.ds(base, ROWS_PT)])

    plsc.subcore_barrier()

    _agg_pipeline(g1_sh, agg_sh, idx_s, pk2.at[c], rows,
                  gsem, ssem0, ssem1)

    plsc.subcore_barrier()
    pltpu.sync_copy(
        agg_sh.at[pl.ds(base, ROWS_PT)],
        agg_out.at[c, pl.ds(base, ROWS_PT)],
    )


@functools.cache
def _sc1_call():
    f32 = jnp.float32
    return pl.kernel(
        _sc1_body,
        out_type=[
            jax.ShapeDtypeStruct((NC, NPAD, H), f32),
            jax.ShapeDtypeStruct((NPAD,), f32),
        ],
        mesh=_mesh(),
        scratch_types=[
            pltpu.VMEM((2, NGR, G), jnp.int32),
            pltpu.VMEM((NGR, G), jnp.int32),
            pltpu.VMEM((G,), f32),
            pltpu.VMEM((ROWS_PT, H), f32),
            pltpu.VMEM((ROWS_PT,), f32),
            pltpu.VMEM((2, G, H), f32),
            pltpu.VMEM_SHARED((NPAD,), f32),
            pltpu.VMEM_SHARED((NPAD, H), f32),
            pltpu.VMEM_SHARED((NPAD, H), f32),
            pltpu.SemaphoreType.DMA,
            pltpu.SemaphoreType.DMA,
            pltpu.SemaphoreType.DMA,
            pltpu.SemaphoreType.DMA,
            pltpu.SemaphoreType.DMA,
        ],
        compiler_params=pltpu.CompilerParams(
            use_tc_tiling_on_sc=False, needs_layout_passes=False),
    )


def _ln(x):
    bits = plsc.bitcast(x, jnp.int32)
    e = lax.shift_right_logical(bits, 23) - 127
    m = plsc.bitcast(
        (bits & jnp.int32(0x7FFFFF)) | jnp.int32(0x3F800000), jnp.float32)
    t = (m - 1.0) / (m + 1.0)
    t2 = t * t
    p = (1.0 / 9.0)
    p = p * t2 + (1.0 / 7.0)
    p = p * t2 + (1.0 / 5.0)
    p = p * t2 + (1.0 / 3.0)
    p = p * t2 + 1.0
    return e.astype(jnp.float32) * 0.6931471805599453 + 2.0 * t * p


def _sc2_body(a1_hbm, h1_hbm, dinv_hbm, b1_hbm, w2_hbm, b2_hbm,
              pk_hbm,
              out_hbm,
              idx_s2, pk2, z_v, t_v, g2p, dinv_v, aT, b1_v, w2_v, b2_v,
              s1_v, s2_v, dinv_sm, out8, rows,
              g2_sh, agg_sh,
              gsem, ssem0, ssem1):
    c = lax.axis_index("c")
    s = lax.axis_index("s")
    base = s * ROWS_PT

    pltpu.sync_copy(pk_hbm.at[s], pk2.at[0])
    pltpu.sync_copy(pk_hbm.at[NS + s], pk2.at[1])
    pltpu.sync_copy(b1_hbm, b1_v)
    pltpu.sync_copy(w2_hbm, w2_v)
    pltpu.sync_copy(b2_hbm, b2_v)
    pltpu.sync_copy(dinv_hbm.at[pl.ds(base, ROWS_PT)], dinv_v)

    def unpack_blk(i, carry):
        sl = pl.ds(i * 16, 16)
        for ch in range(2):
            for j in range(NGR):
                v = pk2[ch, j, sl]
                idx_s2[ch, j, sl] = lax.shift_right_logical(v, 14)
                pk2[ch, j, sl] = v & 16383
        return carry

    lax.fori_loop(0, G // 16, unpack_blk, 0)

    col16 = lax.iota(jnp.int32, 16)
    zvec = jnp.zeros((16,), jnp.float32)

    def zero_blk(b, carry):
        rowi = b * 16 + col16
        for j in range(W2W):
            plsc.store_scatter(g2p, [rowi, jnp.full((16,), j, jnp.int32)],
                               zvec)
        return carry

    lax.fori_loop(0, NBLK, zero_blk, 0)
    pltpu.sync_copy(g2p, agg_sh.at[pl.ds(base, ROWS_PT)])

    pltpu.sync_copy(a1_hbm.at[0, pl.ds(base, ROWS_PT)], z_v)
    pltpu.sync_copy(a1_hbm.at[1, pl.ds(base, ROWS_PT)], t_v)

    def add_blk(b, carry):
        for k in range(16):
            i = b * 16 + k
            z_v[i, :] = z_v[i, :] + t_v[i, :]
        return carry

    lax.fori_loop(0, NBLK, add_blk, 0)
    pltpu.sync_copy(h1_hbm.at[pl.ds(base, ROWS_PT)], t_v)

    def z1_blk(b, carry):
        b1r = b1_v[...]
        dvec = dinv_v[pl.ds(b * 16, 16)]
        for k in range(16):
            i = b * 16 + k
            z = (z_v[i, :] + t_v[i, :] * dvec[k]) * dvec[k] + b1r
            z_v[i, :] = jnp.maximum(z, 0.0)
        return carry

    lax.fori_loop(0, NBLK, z1_blk, 0)

    w2s = [w2_v[k, :] for k in range(16)]

    def mm_blk(b, carry):
        i0 = b * 16
        for n in range(16):
            plsc.store_scatter(
                aT, [col16, jnp.full((16,), n, jnp.int32)], z_v[i0 + n, :])
        dvec = dinv_v[pl.ds(i0, 16)]
        rowi = i0 + col16
        for j in range(C):
            acc = aT[0, :] * w2s[0][j]
            for k in range(1, 16):
                acc = acc + aT[k, :] * w2s[k][j]
            plsc.store_scatter(
                g2p, [rowi, jnp.full((16,), j, jnp.int32)], acc * dvec)
        return carry

    lax.fori_loop(0, NBLK, mm_blk, 0)

    pltpu.sync_copy(g2p, g2_sh.at[pl.ds(base, ROWS_PT)])
    plsc.subcore_barrier()

    _agg_pipeline(g2_sh, agg_sh, idx_s2.at[0], pk2.at[0], rows,
                  gsem, ssem0, ssem1)
    _agg_pipeline(g2_sh, agg_sh, idx_s2.at[1], pk2.at[1], rows,
                  gsem, ssem0, ssem1)
    plsc.subcore_barrier()

    sbase = c * (NPAD // 2) + s * (NPAD // 2 // NS)
    srows = NPAD // 2 // NS
    pltpu.sync_copy(dinv_hbm.at[pl.ds(sbase, srows)], dinv_sm)
    pltpu.sync_copy(agg_sh.at[pl.ds(sbase, srows)], s1_v)
    pltpu.sync_copy(g2_sh.at[pl.ds(sbase, srows)], s2_v)

    l16 = lax.iota(jnp.int32, 16)
    b2c = [b2_v[...][j] for j in range(C)]

    def sm_blk(b, carry):
        rowi = b * 16 + l16
        dvec = dinv_sm[pl.ds(b * 16, 16)]
        zs = []
        for j in range(C):
            colj = jnp.full((16,), j, jnp.int32)
            vj = (plsc.load_gather(s1_v, [rowi, colj])
                  + plsc.load_gather(s2_v, [rowi, colj]))
            zs.append(vj * dvec + b2c[j])
        m = zs[0]
        for j in range(1, C):
            m = jnp.maximum(m, zs[j])
        es = [jnp.exp(z - m) for z in zs]
        ssum = es[0]
        for j in range(1, C):
            ssum = ssum + es[j]
        lse = m + _ln(ssum)
        for j in range(C):
            plsc.store_scatter(
                out8, [rowi, jnp.full((16,), j, jnp.int32)], zs[j] - lse)
        return carry

    lax.fori_loop(0, srows // 16, sm_blk, 0)
    pltpu.sync_copy(out8, out_hbm.at[pl.ds(sbase, srows)])


@functools.cache
def _sc2_call():
    f32 = jnp.float32
    return pl.kernel(
        _sc2_body,
        out_type=jax.ShapeDtypeStruct((NPAD, W2W), f32),
        mesh=_mesh(),
        scratch_types=[
            pltpu.VMEM((2, NGR, G), jnp.int32),
            pltpu.VMEM((2, NGR, G), jnp.int32),
            pltpu.VMEM((ROWS_PT, H), f32),
            pltpu.VMEM((ROWS_PT, H), f32),
            pltpu.VMEM((ROWS_PT, W2W), f32),
            pltpu.VMEM((ROWS_PT,), f32),
            pltpu.VMEM((16, 16), f32),
            pltpu.VMEM((16,), f32),
            pltpu.VMEM((16, 16), f32),
            pltpu.VMEM((16,), f32),
            pltpu.VMEM((NPAD // 2 // NS, W2W), f32),
            pltpu.VMEM((NPAD // 2 // NS, W2W), f32),
            pltpu.VMEM((NPAD // 2 // NS,), f32),
            pltpu.VMEM((NPAD // 2 // NS, W2W), f32),
            pltpu.VMEM((2, G, W2W), f32),
            pltpu.VMEM_SHARED((NPAD, W2W), f32),
            pltpu.VMEM_SHARED((NPAD, W2W), f32),
            pltpu.SemaphoreType.DMA,
            pltpu.SemaphoreType.DMA,
            pltpu.SemaphoreType.DMA,
        ],
        compiler_params=pltpu.CompilerParams(
            use_tc_tiling_on_sc=False, needs_layout_passes=False),
    )


BR = 2048


def _tca_body(x_ref, w_ref, h1_ref):
    h1_ref[...] = jnp.dot(
        x_ref[...], w_ref[...], preferred_element_type=jnp.float32)


def _tcb_body(a0_ref, a1_ref, g2_ref, dinv_ref, b2_ref, out_ref):
    z = dinv_ref[...] * (a0_ref[...] + a1_ref[...] + g2_ref[...]) + b2_ref[...]
    m = jnp.max(z, axis=1, keepdims=True)
    e = jnp.exp(z - m)
    lse = jnp.log(jnp.sum(e, axis=1, keepdims=True))
    out_ref[...] = z - m - lse


def _row_spec(width):
    return pl.BlockSpec((BR, width), lambda i: (i, 0))


def _full_spec(shape):
    return pl.BlockSpec(shape, lambda i: tuple(0 for _ in shape))


def kernel(x, edge_index, W1, b1, W2, b2):
    f32 = jnp.float32
    pk_p = (edge_index[0] * 16384 + edge_index[1]).reshape(NW, NGR, G)
    x_p = jnp.pad(x, ((0, NPAD - N), (0, 0)))
    w2p = jnp.pad(W2, ((0, 0), (0, H - C)))
    b2p = jnp.concatenate([b2, jnp.zeros((16 - C,), f32)])

    grid = (NPAD // BR,)
    h1 = pl.pallas_call(
        _tca_body,
        grid=grid,
        in_specs=[_row_spec(D), _full_spec((D, H))],
        out_specs=_row_spec(H),
        out_shape=jax.ShapeDtypeStruct((NPAD, H), f32),
    )(x_p, W1)

    agg1, dinv2 = _sc1_call()(h1, pk_p)

    out = _sc2_call()(agg1, h1, dinv2, b1, w2p, b2p, pk_p)

    return out[:N, :C]


# speedup vs baseline: 1.0274x; 1.0274x over previous
import functools

import jax
import jax.numpy as jnp
from jax import lax
from jax.experimental import pallas as pl
from jax.experimental.pallas import tpu as pltpu
from jax.experimental.pallas import tpu_sc as plsc

N = 10000
D = 128
H = 16
C = 7
E = 320000

NC = 2
NS = 16
NW = NC * NS

NPAD = 10240
ROWS_PT = NPAD // NS
NBLK = ROWS_PT // 16

EPT = E // NW
G = 2000
NGR = EPT // G
W2W = 8


@functools.cache
def _mesh():
    return plsc.VectorSubcoreMesh(
        core_axis_name="c", subcore_axis_name="s", num_cores=NC, num_subcores=NS
    )


def _newton_rsqrt(x):
    i = plsc.bitcast(x, jnp.int32)
    i = 0x5F3759DF - lax.shift_right_logical(i, 1)
    y = plsc.bitcast(i, jnp.float32)
    for _ in range(3):
        y = y * (1.5 - 0.5 * x * y * y)
    return y


def _agg_pipeline(gsrc_sh, agg_sh, idx_s, idx_d, rows, gsem, ssem0, ssem1):
    def ssem_wait(parity_is_odd, j):
        @pl.when(parity_is_odd == 0)
        def _():
            pltpu.make_async_copy(
                rows.at[0], agg_sh.at[idx_d.at[j]], ssem0).wait()

        @pl.when(parity_is_odd == 1)
        def _():
            pltpu.make_async_copy(
                rows.at[1], agg_sh.at[idx_d.at[j]], ssem1).wait()

    pltpu.async_copy(gsrc_sh.at[idx_s.at[0]], rows.at[0], gsem)

    def step(j, carry):
        buf = lax.rem(j, 2)
        pltpu.make_async_copy(
            gsrc_sh.at[idx_s.at[j]], rows.at[buf], gsem).wait()

        @pl.when(buf == 0)
        def _():
            pltpu.async_copy(
                rows.at[0], agg_sh.at[idx_d.at[j]], ssem0, add=True)

        @pl.when(buf == 1)
        def _():
            pltpu.async_copy(
                rows.at[1], agg_sh.at[idx_d.at[j]], ssem1, add=True)

        @pl.when(j < NGR - 1)
        def _():
            nbuf = lax.rem(j + 1, 2)

            @pl.when(j >= 1)
            def _():
                ssem_wait(nbuf, j - 1)

            pltpu.async_copy(gsrc_sh.at[idx_s.at[j + 1]], rows.at[nbuf], gsem)

        return carry

    lax.fori_loop(0, NGR, step, 0)
    ssem_wait(lax.rem(NGR - 2, 2), NGR - 2)
    ssem_wait(lax.rem(NGR - 1, 2), NGR - 1)


def _sc1_body(h1_hbm, src_hbm, dst_hbm,
              agg_out, dinv_out,
              idx_d2, idx_s, ones_v, h1_v, dinv_v, rows,
              deg_sh, g1_sh, agg_sh,
              hsem, dsem, gsem, ssem0, ssem1):
    c = lax.axis_index("c")
    s = lax.axis_index("s")
    w = c * NS + s
    base = s * ROWS_PT

    def zero_blk(b, carry):
        dinv_v[pl.ds(b * 16, 16)] = jnp.zeros((16,), jnp.float32)
        for k in range(16):
            h1_v[b * 16 + k, :] = jnp.zeros((16,), jnp.float32)
        return carry

    lax.fori_loop(0, NBLK, zero_blk, 0)
    pltpu.sync_copy(h1_v, agg_sh.at[pl.ds(base, ROWS_PT)])
    pltpu.sync_copy(dinv_v, deg_sh.at[pl.ds(base, ROWS_PT)])

    pltpu.async_copy(h1_hbm.at[pl.ds(base, ROWS_PT)], h1_v, hsem)
    pltpu.sync_copy(dst_hbm.at[s], idx_d2.at[0])
    pltpu.sync_copy(dst_hbm.at[NS + s], idx_d2.at[1])
    pltpu.sync_copy(src_hbm.at[w], idx_s)
    for i in range(G // 16):
        ones_v[pl.ds(i * 16, 16)] = jnp.ones((16,), jnp.float32)

    plsc.subcore_barrier()

    descs = []
    for chunk in range(2):
        for j in range(NGR):
            descs.append(pltpu.async_copy(
                ones_v, deg_sh.at[idx_d2.at[chunk, j]], dsem, add=True))
    for dsc in descs:
        dsc.wait()
    plsc.subcore_barrier()

    pltpu.sync_copy(deg_sh.at[pl.ds(base, ROWS_PT)], dinv_v)

    def rsqrt_blk(b, carry):
        x = dinv_v[pl.ds(b * 16, 16)] + 1.0
        dinv_v[pl.ds(b * 16, 16)] = _newton_rsqrt(x)
        return carry

    lax.fori_loop(0, NBLK, rsqrt_blk, 0)

    pltpu.make_async_copy(h1_hbm.at[pl.ds(base, ROWS_PT)], h1_v, hsem).wait()

    def scale_blk(b, carry):
        dvec = dinv_v[pl.ds(b * 16, 16)]
        for k in range(16):
            i = b * 16 + k
            h1_v[i, :] = h1_v[i, :] * dvec[k]
        return carry

    lax.fori_loop(0, NBLK, scale_blk, 0)

    pltpu.sync_copy(h1_v, g1_sh.at[pl.ds(base, ROWS_PT)])

    @pl.when(c == 0)
    def _():
        pltpu.sync_copy(dinv_v, dinv_out.at[pl.ds(base, ROWS_PT)])

    plsc.subcore_barrier()

    _agg_pipeline(g1_sh, agg_sh, idx_s, idx_d2.at[c], rows,
                  gsem, ssem0, ssem1)

    plsc.subcore_barrier()
    pltpu.sync_copy(
        agg_sh.at[pl.ds(base, ROWS_PT)],
        agg_out.at[c, pl.ds(base, ROWS_PT)],
    )


@functools.cache
def _sc1_call():
    f32 = jnp.float32
    return pl.kernel(
        _sc1_body,
        out_type=[
            jax.ShapeDtypeStruct((NC, NPAD, H), f32),
            jax.ShapeDtypeStruct((NPAD,), f32),
        ],
        mesh=_mesh(),
        scratch_types=[
            pltpu.VMEM((2, NGR, G), jnp.int32),
            pltpu.VMEM((NGR, G), jnp.int32),
            pltpu.VMEM((G,), f32),
            pltpu.VMEM((ROWS_PT, H), f32),
            pltpu.VMEM((ROWS_PT,), f32),
            pltpu.VMEM((2, G, H), f32),
            pltpu.VMEM_SHARED((NPAD,), f32),
            pltpu.VMEM_SHARED((NPAD, H), f32),
            pltpu.VMEM_SHARED((NPAD, H), f32),
            pltpu.SemaphoreType.DMA,
            pltpu.SemaphoreType.DMA,
            pltpu.SemaphoreType.DMA,
            pltpu.SemaphoreType.DMA,
            pltpu.SemaphoreType.DMA,
        ],
        compiler_params=pltpu.CompilerParams(
            use_tc_tiling_on_sc=False, needs_layout_passes=False),
    )


def _ln(x):
    bits = plsc.bitcast(x, jnp.int32)
    e = lax.shift_right_logical(bits, 23) - 127
    m = plsc.bitcast(
        (bits & jnp.int32(0x7FFFFF)) | jnp.int32(0x3F800000), jnp.float32)
    t = (m - 1.0) / (m + 1.0)
    t2 = t * t
    p = (1.0 / 9.0)
    p = p * t2 + (1.0 / 7.0)
    p = p * t2 + (1.0 / 5.0)
    p = p * t2 + (1.0 / 3.0)
    p = p * t2 + 1.0
    return e.astype(jnp.float32) * 0.6931471805599453 + 2.0 * t * p


def _sc2_body(a1_hbm, h1_hbm, dinv_hbm, b1_hbm, w2_hbm, b2_hbm,
              src_hbm, dst_hbm,
              out_hbm,
              idx_s2, idx_d2, z_v, t_v, g2p, dinv_v, aT, b1_v, w2_v, b2_v,
              s1_v, s2_v, dinv_sm, out8, rows,
              g2_sh, agg_sh,
              gsem, ssem0, ssem1):
    c = lax.axis_index("c")
    s = lax.axis_index("s")
    base = s * ROWS_PT

    pltpu.sync_copy(src_hbm.at[s], idx_s2.at[0])
    pltpu.sync_copy(src_hbm.at[NS + s], idx_s2.at[1])
    pltpu.sync_copy(dst_hbm.at[s], idx_d2.at[0])
    pltpu.sync_copy(dst_hbm.at[NS + s], idx_d2.at[1])
    pltpu.sync_copy(b1_hbm, b1_v)
    pltpu.sync_copy(w2_hbm, w2_v)
    pltpu.sync_copy(b2_hbm, b2_v)
    pltpu.sync_copy(dinv_hbm.at[pl.ds(base, ROWS_PT)], dinv_v)

    col16 = lax.iota(jnp.int32, 16)
    zvec = jnp.zeros((16,), jnp.float32)

    def zero_blk(b, carry):
        rowi = b * 16 + col16
        for j in range(W2W):
            plsc.store_scatter(g2p, [rowi, jnp.full((16,), j, jnp.int32)],
                               zvec)
        return carry

    lax.fori_loop(0, NBLK, zero_blk, 0)
    pltpu.sync_copy(g2p, agg_sh.at[pl.ds(base, ROWS_PT)])

    pltpu.sync_copy(a1_hbm.at[0, pl.ds(base, ROWS_PT)], z_v)
    pltpu.sync_copy(a1_hbm.at[1, pl.ds(base, ROWS_PT)], t_v)

    def add_blk(b, carry):
        for k in range(16):
            i = b * 16 + k
            z_v[i, :] = z_v[i, :] + t_v[i, :]
        return carry

    lax.fori_loop(0, NBLK, add_blk, 0)
    pltpu.sync_copy(h1_hbm.at[pl.ds(base, ROWS_PT)], t_v)

    def z1_blk(b, carry):
        b1r = b1_v[...]
        dvec = dinv_v[pl.ds(b * 16, 16)]
        for k in range(16):
            i = b * 16 + k
            z = (z_v[i, :] + t_v[i, :] * dvec[k]) * dvec[k] + b1r
            z_v[i, :] = jnp.maximum(z, 0.0)
        return carry

    lax.fori_loop(0, NBLK, z1_blk, 0)

    w2s = [w2_v[k, :] for k in range(16)]

    def mm_blk(b, carry):
        i0 = b * 16
        for n in range(16):
            plsc.store_scatter(
                aT, [col16, jnp.full((16,), n, jnp.int32)], z_v[i0 + n, :])
        dvec = dinv_v[pl.ds(i0, 16)]
        rowi = i0 + col16
        for j in range(C):
            acc = aT[0, :] * w2s[0][j]
            for k in range(1, 16):
                acc = acc + aT[k, :] * w2s[k][j]
            plsc.store_scatter(
                g2p, [rowi, jnp.full((16,), j, jnp.int32)], acc * dvec)
        return carry

    lax.fori_loop(0, NBLK, mm_blk, 0)

    pltpu.sync_copy(g2p, g2_sh.at[pl.ds(base, ROWS_PT)])
    plsc.subcore_barrier()

    _agg_pipeline(g2_sh, agg_sh, idx_s2.at[0], idx_d2.at[0], rows,
                  gsem, ssem0, ssem1)
    _agg_pipeline(g2_sh, agg_sh, idx_s2.at[1], idx_d2.at[1], rows,
                  gsem, ssem0, ssem1)
    plsc.subcore_barrier()

    sbase = c * (NPAD // 2) + s * (NPAD // 2 // NS)
    srows = NPAD // 2 // NS
    pltpu.sync_copy(dinv_hbm.at[pl.ds(sbase, srows)], dinv_sm)
    pltpu.sync_copy(agg_sh.at[pl.ds(sbase, srows)], s1_v)
    pltpu.sync_copy(g2_sh.at[pl.ds(sbase, srows)], s2_v)

    l16 = lax.iota(jnp.int32, 16)
    b2c = [b2_v[...][j] for j in range(C)]

    def sm_blk(b, carry):
        rowi = b * 16 + l16
        dvec = dinv_sm[pl.ds(b * 16, 16)]
        zs = []
        for j in range(C):
            colj = jnp.full((16,), j, jnp.int32)
            vj = (plsc.load_gather(s1_v, [rowi, colj])
                  + plsc.load_gather(s2_v, [rowi, colj]))
            zs.append(vj * dvec + b2c[j])
        m = zs[0]
        for j in range(1, C):
            m = jnp.maximum(m, zs[j])
        es = [jnp.exp(z - m) for z in zs]
        ssum = es[0]
        for j in range(1, C):
            ssum = ssum + es[j]
        lse = m + _ln(ssum)
        for j in range(C):
            plsc.store_scatter(
                out8, [rowi, jnp.full((16,), j, jnp.int32)], zs[j] - lse)
        return carry

    lax.fori_loop(0, srows // 16, sm_blk, 0)
    pltpu.sync_copy(out8, out_hbm.at[pl.ds(sbase, srows)])


@functools.cache
def _sc2_call():
    f32 = jnp.float32
    return pl.kernel(
        _sc2_body,
        out_type=jax.ShapeDtypeStruct((NPAD, W2W), f32),
        mesh=_mesh(),
        scratch_types=[
            pltpu.VMEM((2, NGR, G), jnp.int32),
            pltpu.VMEM((2, NGR, G), jnp.int32),
            pltpu.VMEM((ROWS_PT, H), f32),
            pltpu.VMEM((ROWS_PT, H), f32),
            pltpu.VMEM((ROWS_PT, W2W), f32),
            pltpu.VMEM((ROWS_PT,), f32),
            pltpu.VMEM((16, 16), f32),
            pltpu.VMEM((16,), f32),
            pltpu.VMEM((16, 16), f32),
            pltpu.VMEM((16,), f32),
            pltpu.VMEM((NPAD // 2 // NS, W2W), f32),
            pltpu.VMEM((NPAD // 2 // NS, W2W), f32),
            pltpu.VMEM((NPAD // 2 // NS,), f32),
            pltpu.VMEM((NPAD // 2 // NS, W2W), f32),
            pltpu.VMEM((2, G, W2W), f32),
            pltpu.VMEM_SHARED((NPAD, W2W), f32),
            pltpu.VMEM_SHARED((NPAD, W2W), f32),
            pltpu.SemaphoreType.DMA,
            pltpu.SemaphoreType.DMA,
            pltpu.SemaphoreType.DMA,
        ],
        compiler_params=pltpu.CompilerParams(
            use_tc_tiling_on_sc=False, needs_layout_passes=False),
    )


BR = 2048


def _tca_body(x_ref, w_ref, h1_ref):
    h1_ref[...] = jnp.dot(
        x_ref[...], w_ref[...], preferred_element_type=jnp.float32)


def _tcb_body(a0_ref, a1_ref, g2_ref, dinv_ref, b2_ref, out_ref):
    z = dinv_ref[...] * (a0_ref[...] + a1_ref[...] + g2_ref[...]) + b2_ref[...]
    m = jnp.max(z, axis=1, keepdims=True)
    e = jnp.exp(z - m)
    lse = jnp.log(jnp.sum(e, axis=1, keepdims=True))
    out_ref[...] = z - m - lse


def _row_spec(width):
    return pl.BlockSpec((BR, width), lambda i: (i, 0))


def _full_spec(shape):
    return pl.BlockSpec(shape, lambda i: tuple(0 for _ in shape))


def kernel(x, edge_index, W1, b1, W2, b2):
    f32 = jnp.float32
    src_p = edge_index[0].reshape(NW, NGR, G)
    dst_p = edge_index[1].reshape(NW, NGR, G)
    x_p = jnp.pad(x, ((0, NPAD - N), (0, 0)))
    w2p = jnp.pad(W2, ((0, 0), (0, H - C)))
    b2p = jnp.concatenate([b2, jnp.zeros((16 - C,), f32)])

    grid = (NPAD // BR,)
    h1 = pl.pallas_call(
        _tca_body,
        grid=grid,
        in_specs=[_row_spec(D), _full_spec((D, H))],
        out_specs=_row_spec(H),
        out_shape=jax.ShapeDtypeStruct((NPAD, H), f32),
    )(x_p, W1)

    agg1, dinv2 = _sc1_call()(h1, src_p, dst_p)

    out = _sc2_call()(agg1, h1, dinv2, b1, w2p, b2p, src_p, dst_p)

    return out[:N, :C]
